# baseline (speedup 1.0000x reference)
import functools

import jax
import jax.numpy as jnp
from jax import lax
from jax.experimental import pallas as pl
from jax.experimental.pallas import tpu as pltpu
from jax.experimental.pallas import tpu_sc as plsc

D_MODEL = 64
VOCAB = 1000000
BATCH = 4096
HIST = 50

NC = 2
NS = 16
NW = NC * NS

GRP = BATCH // NW
N_STAGE = HIST
N_CHUNK = GRP // 16

SCALE = 8.0


def _mesh():
    return plsc.VectorSubcoreMesh(core_axis_name="c", subcore_axis_name="s")


@functools.partial(
    pl.kernel,
    mesh=_mesh(),
    out_type=jax.ShapeDtypeStruct((HIST, D_MODEL, BATCH), jnp.float32),
    scratch_types=[
        pltpu.VMEM((N_STAGE, GRP), jnp.int32),
        pltpu.VMEM((2, GRP), jnp.int32),
        pltpu.VMEM((2, GRP, 2 * D_MODEL), jnp.float32),
        pltpu.VMEM((2, D_MODEL, GRP), jnp.float32),
        pltpu.SemaphoreType.DMA,
        pltpu.SemaphoreType.DMA,
    ],
    compiler_params=pltpu.CompilerParams(needs_layout_passes=False),
)
def _gather_scale(idx_hbm, table_hbm, out_hbm, idx_v, idxp_v, buf, obuf,
                  sem0, sem1):
    wid = lax.axis_index("s") * NC + lax.axis_index("c")
    b0 = pl.multiple_of(wid * GRP, GRP)
    sems = (sem0, sem1)
    pltpu.sync_copy(idx_hbm.at[:, pl.ds(b0, GRP)], idx_v)

    lanes = lax.iota(jnp.int32, 16)

    def pair_indices(st, p):
        def c_body(c, carry):
            v = idx_v[st, pl.ds(c * 16, 16)]
            idxp_v[p, pl.ds(c * 16, 16)] = lax.shift_right_logical(v, 1)
            return carry
        lax.fori_loop(0, N_CHUNK, c_body, 0)

    def fire(p):
        pltpu.async_copy(table_hbm.at[idxp_v.at[p]], buf.at[p], sems[p])

    def drain(p):
        pltpu.make_async_copy(
            table_hbm.at[idxp_v.at[p]], buf.at[p], sems[p]).wait()

    def select_scale_store(st, p):
        def c_body(c, carry):
            kvec = c * 16 + lanes
            parity = lax.bitwise_and(idx_v[st, pl.ds(c * 16, 16)], 1)
            scol = parity * D_MODEL
            for j in range(D_MODEL):
                v = plsc.load_gather(buf.at[p], [kvec, scol])
                obuf[p, j, pl.ds(c * 16, 16)] = v * SCALE
                if j != D_MODEL - 1:
                    scol = scol + 1
            return carry
        lax.fori_loop(0, N_CHUNK, c_body, 0)
        pltpu.sync_copy(obuf.at[p], out_hbm.at[st, :, pl.ds(b0, GRP)])

    pair_indices(0, 0)
    fire(0)

    def pair_body(u, carry):
        for q in range(2):
            st = 2 * u + q
            @pl.when(st + 1 < N_STAGE)
            def _fire_next():
                pair_indices(st + 1, 1 - q)
                fire(1 - q)
            drain(q)
            select_scale_store(st, q)
        return carry
    lax.fori_loop(0, N_STAGE // 2, pair_body, 0)


def kernel(x, W):
    idx = jnp.transpose(x.reshape(BATCH, HIST)).astype(jnp.int32)
    table = jnp.concatenate([W[0::2], W[1::2]], axis=1)
    out = _gather_scale(idx, table)
    return jnp.transpose(out, (2, 0, 1))


# speedup vs baseline: 12.5605x; 12.5605x over previous
import functools

import jax
import jax.numpy as jnp
from jax import lax
from jax.experimental import pallas as pl
from jax.experimental.pallas import tpu as pltpu
from jax.experimental.pallas import tpu_sc as plsc

D_MODEL = 64
VOCAB = 1000000
BATCH = 4096
HIST = 50

NC = 2
NS = 16
NW = NC * NS

GRP = BATCH // NW
N_STAGE = HIST
PADW = 2 * D_MODEL

SCALE = 8.0


def _mesh():
    return plsc.VectorSubcoreMesh(core_axis_name="c", subcore_axis_name="s")


@functools.partial(
    pl.kernel,
    mesh=_mesh(),
    out_type=jax.ShapeDtypeStruct((HIST, BATCH, PADW), jnp.float32),
    scratch_types=[
        pltpu.VMEM((N_STAGE, GRP), jnp.int32),
        pltpu.VMEM((2, GRP, PADW), jnp.float32),
        pltpu.SemaphoreType.DMA,
        pltpu.SemaphoreType.DMA,
    ],
    compiler_params=pltpu.CompilerParams(use_tc_tiling_on_sc=False),
)
def _gather_scale(idx_hbm, table_hbm, out_hbm, idx_v, buf, sem0, sem1):
    wid = lax.axis_index("s") * NC + lax.axis_index("c")
    b0 = wid * GRP
    sems = (sem0, sem1)
    pltpu.sync_copy(idx_hbm.at[:, pl.ds(b0, GRP)], idx_v)

    def fire(st, p):
        pltpu.async_copy(table_hbm.at[idx_v.at[st]], buf.at[p], sems[p])

    def drain(st, p):
        pltpu.make_async_copy(
            table_hbm.at[idx_v.at[st]], buf.at[p], sems[p]).wait()

    def scale_store(st, p):
        def r_body(r, carry):
            for q in range(D_MODEL // 16):
                sl = pl.ds(q * 16, 16)
                buf[p, r, sl] = buf[p, r, sl] * SCALE
            return carry
        lax.fori_loop(0, GRP, r_body, 0)
        pltpu.sync_copy(buf.at[p], out_hbm.at[st, pl.ds(b0, GRP)])

    fire(0, 0)

    def pair_body(u, carry):
        for q in range(2):
            st = 2 * u + q
            @pl.when(st + 1 < N_STAGE)
            def _fire_next():
                fire(st + 1, 1 - q)
            drain(st, q)
            scale_store(st, q)
        return carry
    lax.fori_loop(0, N_STAGE // 2, pair_body, 0)


def kernel(x, W):
    idx = jnp.transpose(x.reshape(BATCH, HIST)).astype(jnp.int32)
    table = jnp.pad(W, ((0, 0), (0, D_MODEL)))
    out = _gather_scale(idx, table)
    return jnp.transpose(out[:, :, :D_MODEL], (1, 0, 2))


# speedup vs baseline: 19.1549x; 1.5250x over previous
import functools

import jax
import jax.numpy as jnp
from jax import lax
from jax.experimental import pallas as pl
from jax.experimental.pallas import tpu as pltpu
from jax.experimental.pallas import tpu_sc as plsc

D_MODEL = 64
VOCAB = 1000000
BATCH = 4096
HIST = 50

NC = 2
NS = 16
NW = NC * NS

GRP = BATCH // NW
N_STAGE = HIST

SCALE = 8.0


def _mesh():
    return plsc.VectorSubcoreMesh(core_axis_name="c", subcore_axis_name="s")


@functools.partial(
    pl.kernel,
    mesh=_mesh(),
    out_type=jax.ShapeDtypeStruct((HIST, BATCH, D_MODEL), jnp.float32),
    scratch_types=[
        pltpu.VMEM((N_STAGE, GRP), jnp.int32),
        pltpu.VMEM((2, GRP, D_MODEL), jnp.float32),
        pltpu.SemaphoreType.DMA,
        pltpu.SemaphoreType.DMA,
    ],
    compiler_params=pltpu.CompilerParams(needs_layout_passes=False),
)
def _gather_scale(idx_hbm, table_hbm, out_hbm, idx_v, buf, sem0, sem1):
    wid = lax.axis_index("s") * NC + lax.axis_index("c")
    b0 = pl.multiple_of(wid * GRP, GRP)
    sems = (sem0, sem1)
    pltpu.sync_copy(idx_hbm.at[:, pl.ds(b0, GRP)], idx_v)

    def fire(st, p):
        def c_body(c, carry):
            chunk = idx_v[st, pl.ds(c * 16, 16)]
            for i in range(16):
                pltpu.async_copy(
                    table_hbm.at[chunk[i]], buf.at[p, c * 16 + i], sems[p])
            return carry
        lax.fori_loop(0, GRP // 16, c_body, 0)

    def drain(p):
        pltpu.make_async_copy(
            table_hbm.at[pl.ds(0, GRP)], buf.at[p], sems[p]).wait()

    def scale_store(st, p):
        def r_body(r, carry):
            for q in range(D_MODEL // 16):
                sl = pl.ds(q * 16, 16)
                buf[p, r, sl] = buf[p, r, sl] * SCALE
            return carry
        lax.fori_loop(0, GRP, r_body, 0)
        pltpu.sync_copy(buf.at[p], out_hbm.at[st, pl.ds(b0, GRP)])

    fire(0, 0)

    def pair_body(u, carry):
        for q in range(2):
            st = 2 * u + q
            @pl.when(st + 1 < N_STAGE)
            def _fire_next():
                fire(st + 1, 1 - q)
            drain(q)
            scale_store(st, q)
        return carry
    lax.fori_loop(0, N_STAGE // 2, pair_body, 0)


def kernel(x, W):
    idx = jnp.transpose(x.reshape(BATCH, HIST)).astype(jnp.int32)
    out = _gather_scale(idx, W)
    return jnp.transpose(out, (1, 0, 2))


# speedup vs baseline: 26.2634x; 1.3711x over previous
import functools

import jax
import jax.numpy as jnp
from jax import lax
from jax.experimental import pallas as pl
from jax.experimental.pallas import tpu as pltpu
from jax.experimental.pallas import tpu_sc as plsc

D_MODEL = 64
VOCAB = 1000000
BATCH = 4096
HIST = 50

NC = 2
NS = 16
NW = NC * NS

GRP = BATCH // NW
N_STAGE = HIST

SCALE = 8.0


def _mesh():
    return plsc.VectorSubcoreMesh(core_axis_name="c", subcore_axis_name="s")


@functools.partial(
    pl.kernel,
    mesh=_mesh(),
    out_type=jax.ShapeDtypeStruct((HIST, NW, GRP // 8, 8, D_MODEL),
                                  jnp.float32),
    scratch_types=[
        pltpu.VMEM((N_STAGE, GRP), jnp.int32),
        pltpu.VMEM((2, GRP // 8, 8, D_MODEL), jnp.float32),
        pltpu.SemaphoreType.DMA,
        pltpu.SemaphoreType.DMA,
    ],
    compiler_params=pltpu.CompilerParams(needs_layout_passes=False),
)
def _gather_scale(idx_hbm, table_hbm, out_hbm, idx_v, buf, sem0, sem1):
    wid = lax.axis_index("s") * NC + lax.axis_index("c")
    b0 = pl.multiple_of(wid * GRP, GRP)
    sems = (sem0, sem1)
    pltpu.sync_copy(idx_hbm.at[:, pl.ds(b0, GRP)], idx_v)

    def fire(st, p):
        def c_body(c, carry):
            chunk = idx_v[st, pl.ds(c * 16, 16)]
            for i in range(16):
                v = chunk[i]
                pltpu.async_copy(
                    table_hbm.at[lax.shift_right_logical(v, 3),
                                 lax.bitwise_and(v, 7)],
                    buf.at[p, 2 * c + i // 8, i % 8],
                    sems[p])
            return carry
        lax.fori_loop(0, GRP // 16, c_body, 0)

    def drain(p):
        pltpu.make_async_copy(
            table_hbm.at[pl.ds(0, GRP // 8)], buf.at[p], sems[p]).wait()

    def scale_store(st, p):
        def r_body(r, carry):
            for j in range(8):
                for q in range(D_MODEL // 16):
                    sl = pl.ds(q * 16, 16)
                    buf[p, r, j, sl] = buf[p, r, j, sl] * SCALE
            return carry
        lax.fori_loop(0, GRP // 8, r_body, 0)
        pltpu.sync_copy(buf.at[p], out_hbm.at[st, wid])

    fire(0, 0)

    def pair_body(u, carry):
        for q in range(2):
            st = 2 * u + q
            @pl.when(st + 1 < N_STAGE)
            def _fire_next():
                fire(st + 1, 1 - q)
            drain(q)
            scale_store(st, q)
        return carry
    lax.fori_loop(0, N_STAGE // 2, pair_body, 0)


def kernel(x, W):
    idx = jnp.transpose(x.reshape(BATCH, HIST)).astype(jnp.int32)
    table = W.reshape(VOCAB // 8, 8, D_MODEL)
    out = _gather_scale(idx, table)
    return jnp.transpose(out.reshape(HIST, BATCH, D_MODEL), (1, 0, 2))
